# baseline (speedup 1.0000x reference)
import functools

import jax
import jax.numpy as jnp
from jax import lax
from jax.experimental import pallas as pl
from jax.experimental.pallas import tpu as pltpu
from jax.experimental.pallas import tpu_sc as plsc

B = 16384
L = 50
EMB = 64
HID = 64
NCORES = 2
NSUB = 16
NW = NCORES * NSUB
ROWS_PER_W = (2 * B) // NW
CB = 8
NCH = ROWS_PER_W // CB
LANES = 16
LP = 56


def _sc_pool_body(ids_hbm, table_hbm, out_hbm,
                  idx_v, rows0, rows1, pooled0, pooled1,
                  sem_g0, sem_g1, sem_s0, sem_s1):
    rows_v = (rows0, rows1)
    pooled_v = (pooled0, pooled1)
    sem_g = (sem_g0, sem_g1)
    sem_s = (sem_s0, sem_s1)

    wid = lax.axis_index("s") * NCORES + lax.axis_index("c")
    base = wid * ROWS_PER_W
    brow = base % B
    col = (base // B) * EMB

    pltpu.sync_copy(ids_hbm.at[pl.ds(base * LP, ROWS_PER_W * LP)], idx_v)

    def fire(c, b):
        for r in range(CB):
            g = c * CB + r
            pltpu.async_copy(table_hbm.at[idx_v.at[pl.ds(g * LP, L)]],
                             rows_v[b].at[pl.ds(r * L, L)], sem_g[b])

    def drain(c, b):
        for r in range(CB):
            g = c * CB + r
            pltpu.make_async_copy(table_hbm.at[idx_v.at[pl.ds(g * LP, L)]],
                                  rows_v[b].at[pl.ds(r * L, L)],
                                  sem_g[b]).wait()
        @pl.when(c >= 2)
        def _():
            pltpu.make_async_copy(pooled_v[b],
                                  out_hbm.at[pl.ds(brow, CB), pl.ds(col, EMB)],
                                  sem_s[b]).wait()
        scale = jnp.float32(1.0 / L)
        for r in range(CB):
            rbase = r * L

            def lbody(l, acc):
                a0, a1, a2, a3 = acc
                row = rbase + l
                return (a0 + rows_v[b][row, pl.ds(0, LANES)],
                        a1 + rows_v[b][row, pl.ds(LANES, LANES)],
                        a2 + rows_v[b][row, pl.ds(2 * LANES, LANES)],
                        a3 + rows_v[b][row, pl.ds(3 * LANES, LANES)])

            z = jnp.zeros((LANES,), jnp.float32)
            a0, a1, a2, a3 = lax.fori_loop(0, L, lbody, (z, z, z, z),
                                           unroll=10)
            pooled_v[b][r, pl.ds(0, LANES)] = a0 * scale
            pooled_v[b][r, pl.ds(LANES, LANES)] = a1 * scale
            pooled_v[b][r, pl.ds(2 * LANES, LANES)] = a2 * scale
            pooled_v[b][r, pl.ds(3 * LANES, LANES)] = a3 * scale
        pltpu.async_copy(pooled_v[b],
                         out_hbm.at[pl.ds(brow + c * CB, CB), pl.ds(col, EMB)],
                         sem_s[b])

    fire(0, 0)

    def body(c):
        fire(c + 1, 1)
        drain(c, 0)

        @pl.when(c + 2 < NCH)
        def _():
            fire(c + 2, 0)

        drain(c + 1, 1)

    pl.loop(0, NCH, step=2)(body)

    for b in range(2):
        pltpu.make_async_copy(pooled_v[b],
                              out_hbm.at[pl.ds(brow, CB), pl.ds(col, EMB)],
                              sem_s[b]).wait()


def _sc_pool(ids, table):
    mesh = plsc.VectorSubcoreMesh(core_axis_name="c", subcore_axis_name="s",
                                  num_cores=NCORES, num_subcores=NSUB)
    return pl.kernel(
        _sc_pool_body,
        out_type=jax.ShapeDtypeStruct((B, 2 * EMB), jnp.float32),
        mesh=mesh,
        compiler_params=pltpu.CompilerParams(use_tc_tiling_on_sc=False),
        scratch_types=[
            pltpu.VMEM((ROWS_PER_W * LP,), jnp.int32),
            pltpu.VMEM((CB * L, EMB), jnp.float32),
            pltpu.VMEM((CB * L, EMB), jnp.float32),
            pltpu.VMEM((CB, EMB), jnp.float32),
            pltpu.VMEM((CB, EMB), jnp.float32),
            pltpu.SemaphoreType.DMA,
            pltpu.SemaphoreType.DMA,
            pltpu.SemaphoreType.DMA,
            pltpu.SemaphoreType.DMA,
        ],
    )(ids, table)


VOCAB = 1000000
SPLIT = 512000
DEPAD_BN = 10240
DEPAD_G = SPLIT // DEPAD_BN
DEPAD_LAST = (VOCAB - 1) // DEPAD_BN


def _depad_body(ta_ref, tb_ref, out_ref):
    stacked = jnp.concatenate([ta_ref[...], tb_ref[...]], axis=0)
    out_ref[...] = stacked.T


def _depad(tableT):
    return pl.pallas_call(
        _depad_body,
        grid=(DEPAD_G,),
        in_specs=[
            pl.BlockSpec((EMB, DEPAD_BN), lambda i: (0, i)),
            pl.BlockSpec((EMB, DEPAD_BN),
                         lambda i: (0, jnp.minimum(i + DEPAD_G, DEPAD_LAST))),
        ],
        out_specs=pl.BlockSpec((DEPAD_BN, 2 * EMB), lambda i: (i, 0)),
        out_shape=jax.ShapeDtypeStruct((SPLIT, 2 * EMB), jnp.float32),
    )(tableT, tableT)


BM = 2048


def _mlp_body(ph_ref, w1t_ref, b1_ref, w2t_ref, b2_ref, out_ref):
    ph = ph_ref[...]
    p = ph[:, :EMB]
    h = ph[:, EMB:]
    f = jnp.concatenate([p, h, jnp.abs(p - h)], axis=1)
    x = jnp.dot(f, w1t_ref[...], preferred_element_type=jnp.float32)
    x = jnp.maximum(x + b1_ref[...], 0.0)
    logits = jnp.dot(x, w2t_ref[...], preferred_element_type=jnp.float32)
    logits = logits + b2_ref[...]
    m = jnp.max(logits, axis=1, keepdims=True)
    lse = m + jnp.log(jnp.sum(jnp.exp(logits - m), axis=1, keepdims=True))
    out_ref[...] = (logits - lse).T


def _mlp(pooled, w1t, b1, w2t, b2):
    grid = (B // BM,)
    return pl.pallas_call(
        _mlp_body,
        grid=grid,
        in_specs=[
            pl.BlockSpec((BM, 2 * EMB), lambda i: (i, 0)),
            pl.BlockSpec((3 * EMB, HID), lambda i: (0, 0)),
            pl.BlockSpec((1, HID), lambda i: (0, 0)),
            pl.BlockSpec((HID, 2), lambda i: (0, 0)),
            pl.BlockSpec((1, 2), lambda i: (0, 0)),
        ],
        out_specs=pl.BlockSpec((2, BM), lambda i: (0, i)),
        out_shape=jax.ShapeDtypeStruct((2, B), jnp.float32),
    )(pooled, w1t, b1, w2t, b2)


def kernel(premise, hypothesis, table, W1, b1, W2, b2):
    ids = jnp.concatenate([premise, hypothesis], axis=0).astype(jnp.int32)
    ids = jnp.where(ids < SPLIT, 2 * ids, 2 * (ids - SPLIT) + 1)
    ids = jnp.pad(ids, ((0, 0), (0, LP - L))).reshape(-1)
    table_lin = _depad(table.T).reshape(2 * SPLIT, EMB)
    pooled = _sc_pool(ids, table_lin)
    w1t = W1.T
    w2t = W2.T
    b1r = b1.reshape(1, HID)
    b2r = b2.reshape(1, 2)
    return _mlp(pooled, w1t, b1r, w2t, b2r).T


# speedup vs baseline: 1.0719x; 1.0719x over previous
import functools

import jax
import jax.numpy as jnp
from jax import lax
from jax.experimental import pallas as pl
from jax.experimental.pallas import tpu as pltpu
from jax.experimental.pallas import tpu_sc as plsc

B = 16384
L = 50
EMB = 64
HID = 64
NCORES = 2
NSUB = 16
NW = NCORES * NSUB
ROWS_PER_W = (2 * B) // NW
CB = 8
NCH = ROWS_PER_W // CB
LANES = 16
LP = 56


def _sc_pool_body(ids_hbm, table_hbm, out_hbm,
                  idx_v, rows0, rows1, pooled0, pooled1,
                  sem_g0, sem_g1, sem_s0, sem_s1):
    rows_v = (rows0, rows1)
    pooled_v = (pooled0, pooled1)
    sem_g = (sem_g0, sem_g1)
    sem_s = (sem_s0, sem_s1)

    wid = lax.axis_index("s") * NCORES + lax.axis_index("c")
    base = wid * ROWS_PER_W
    brow = base % B
    col = (base // B) * EMB

    pltpu.sync_copy(ids_hbm.at[pl.ds(base * LP, ROWS_PER_W * LP)], idx_v)

    def fire(c, b):
        for r in range(CB):
            g = c * CB + r
            pltpu.async_copy(table_hbm.at[idx_v.at[pl.ds(g * LP, L)]],
                             rows_v[b].at[pl.ds(r * L, L)], sem_g[b])

    def drain(c, b):
        for r in range(CB):
            g = c * CB + r
            pltpu.make_async_copy(table_hbm.at[idx_v.at[pl.ds(g * LP, L)]],
                                  rows_v[b].at[pl.ds(r * L, L)],
                                  sem_g[b]).wait()
        @pl.when(c >= 2)
        def _():
            pltpu.make_async_copy(pooled_v[b],
                                  out_hbm.at[pl.ds(brow, CB), pl.ds(col, EMB)],
                                  sem_s[b]).wait()
        scale = jnp.float32(1.0 / L)
        for r in range(CB):
            rbase = r * L

            def lbody(l, acc):
                a0, a1, a2, a3 = acc
                row = rbase + l
                return (a0 + rows_v[b][row, pl.ds(0, LANES)],
                        a1 + rows_v[b][row, pl.ds(LANES, LANES)],
                        a2 + rows_v[b][row, pl.ds(2 * LANES, LANES)],
                        a3 + rows_v[b][row, pl.ds(3 * LANES, LANES)])

            z = jnp.zeros((LANES,), jnp.float32)
            a0, a1, a2, a3 = lax.fori_loop(0, L, lbody, (z, z, z, z),
                                           unroll=5)
            pooled_v[b][r, pl.ds(0, LANES)] = a0 * scale
            pooled_v[b][r, pl.ds(LANES, LANES)] = a1 * scale
            pooled_v[b][r, pl.ds(2 * LANES, LANES)] = a2 * scale
            pooled_v[b][r, pl.ds(3 * LANES, LANES)] = a3 * scale
        pltpu.async_copy(pooled_v[b],
                         out_hbm.at[pl.ds(brow + c * CB, CB), pl.ds(col, EMB)],
                         sem_s[b])

    fire(0, 0)

    def body(c):
        fire(c + 1, 1)
        drain(c, 0)

        @pl.when(c + 2 < NCH)
        def _():
            fire(c + 2, 0)

        drain(c + 1, 1)

    pl.loop(0, NCH, step=2)(body)

    for b in range(2):
        pltpu.make_async_copy(pooled_v[b],
                              out_hbm.at[pl.ds(brow, CB), pl.ds(col, EMB)],
                              sem_s[b]).wait()


def _sc_pool(ids, table):
    mesh = plsc.VectorSubcoreMesh(core_axis_name="c", subcore_axis_name="s",
                                  num_cores=NCORES, num_subcores=NSUB)
    return pl.kernel(
        _sc_pool_body,
        out_type=jax.ShapeDtypeStruct((B, 2 * EMB), jnp.float32),
        mesh=mesh,
        compiler_params=pltpu.CompilerParams(use_tc_tiling_on_sc=False),
        scratch_types=[
            pltpu.VMEM((ROWS_PER_W * LP,), jnp.int32),
            pltpu.VMEM((CB * L, EMB), jnp.float32),
            pltpu.VMEM((CB * L, EMB), jnp.float32),
            pltpu.VMEM((CB, EMB), jnp.float32),
            pltpu.VMEM((CB, EMB), jnp.float32),
            pltpu.SemaphoreType.DMA,
            pltpu.SemaphoreType.DMA,
            pltpu.SemaphoreType.DMA,
            pltpu.SemaphoreType.DMA,
        ],
    )(ids, table)


VOCAB = 1000000
SPLIT = 512000
DEPAD_BN = 10240
DEPAD_G = SPLIT // DEPAD_BN
DEPAD_LAST = (VOCAB - 1) // DEPAD_BN


def _depad_body(ta_ref, tb_ref, out_ref):
    stacked = jnp.concatenate([ta_ref[...], tb_ref[...]], axis=0)
    out_ref[...] = stacked.T


def _depad(tableT):
    return pl.pallas_call(
        _depad_body,
        grid=(DEPAD_G,),
        in_specs=[
            pl.BlockSpec((EMB, DEPAD_BN), lambda i: (0, i)),
            pl.BlockSpec((EMB, DEPAD_BN),
                         lambda i: (0, jnp.minimum(i + DEPAD_G, DEPAD_LAST))),
        ],
        out_specs=pl.BlockSpec((DEPAD_BN, 2 * EMB), lambda i: (i, 0)),
        out_shape=jax.ShapeDtypeStruct((SPLIT, 2 * EMB), jnp.float32),
    )(tableT, tableT)


BM = 2048


def _mlp_body(ph_ref, w1t_ref, b1_ref, w2t_ref, b2_ref, out_ref):
    ph = ph_ref[...]
    p = ph[:, :EMB]
    h = ph[:, EMB:]
    f = jnp.concatenate([p, h, jnp.abs(p - h)], axis=1)
    x = jnp.dot(f, w1t_ref[...], preferred_element_type=jnp.float32)
    x = jnp.maximum(x + b1_ref[...], 0.0)
    logits = jnp.dot(x, w2t_ref[...], preferred_element_type=jnp.float32)
    logits = logits + b2_ref[...]
    m = jnp.max(logits, axis=1, keepdims=True)
    lse = m + jnp.log(jnp.sum(jnp.exp(logits - m), axis=1, keepdims=True))
    out_ref[...] = (logits - lse).T


def _mlp(pooled, w1t, b1, w2t, b2):
    grid = (B // BM,)
    return pl.pallas_call(
        _mlp_body,
        grid=grid,
        in_specs=[
            pl.BlockSpec((BM, 2 * EMB), lambda i: (i, 0)),
            pl.BlockSpec((3 * EMB, HID), lambda i: (0, 0)),
            pl.BlockSpec((1, HID), lambda i: (0, 0)),
            pl.BlockSpec((HID, 2), lambda i: (0, 0)),
            pl.BlockSpec((1, 2), lambda i: (0, 0)),
        ],
        out_specs=pl.BlockSpec((2, BM), lambda i: (0, i)),
        out_shape=jax.ShapeDtypeStruct((2, B), jnp.float32),
    )(pooled, w1t, b1, w2t, b2)


def kernel(premise, hypothesis, table, W1, b1, W2, b2):
    ids = jnp.concatenate([premise, hypothesis], axis=0).astype(jnp.int32)
    ids = jnp.where(ids < SPLIT, 2 * ids, 2 * (ids - SPLIT) + 1)
    ids = jnp.pad(ids, ((0, 0), (0, LP - L))).reshape(-1)
    table_lin = _depad(table.T).reshape(2 * SPLIT, EMB)
    pooled = _sc_pool(ids, table_lin)
    w1t = W1.T
    w2t = W2.T
    b1r = b1.reshape(1, HID)
    b2r = b2.reshape(1, 2)
    return _mlp(pooled, w1t, b1r, w2t, b2r).T


# speedup vs baseline: 1.0854x; 1.0125x over previous
import functools

import jax
import jax.numpy as jnp
from jax import lax
from jax.experimental import pallas as pl
from jax.experimental.pallas import tpu as pltpu
from jax.experimental.pallas import tpu_sc as plsc

B = 16384
L = 50
EMB = 64
HID = 64
NCORES = 2
NSUB = 16
NW = NCORES * NSUB
ROWS_PER_W = (2 * B) // NW
CB = 8
NCH = ROWS_PER_W // CB
LANES = 16
LP = 56


def _sc_pool_body(ids_hbm, table_hbm, out_hbm,
                  idx_v, rows0, rows1, pooled0, pooled1,
                  sem_g0, sem_g1, sem_s0, sem_s1):
    rows_v = (rows0, rows1)
    pooled_v = (pooled0, pooled1)
    sem_g = (sem_g0, sem_g1)
    sem_s = (sem_s0, sem_s1)

    wid = lax.axis_index("s") * NCORES + lax.axis_index("c")
    base = wid * ROWS_PER_W
    brow = base % B
    col = (base // B) * EMB

    pltpu.sync_copy(ids_hbm.at[pl.ds(base * LP, ROWS_PER_W * LP)], idx_v)

    def fire(c, b):
        for r in range(CB):
            g = c * CB + r
            pltpu.async_copy(table_hbm.at[idx_v.at[pl.ds(g * LP, L)]],
                             rows_v[b].at[pl.ds(r * L, L)], sem_g[b])

    def drain(c, b):
        pltpu.make_async_copy(table_hbm.at[pl.ds(0, CB * L)],
                              rows_v[b], sem_g[b]).wait()
        @pl.when(c >= 2)
        def _():
            pltpu.make_async_copy(pooled_v[b],
                                  out_hbm.at[pl.ds(brow, CB), pl.ds(col, EMB)],
                                  sem_s[b]).wait()
        scale = jnp.float32(1.0 / L)
        for r in range(CB):
            rbase = r * L

            def lbody(l, acc):
                a0, a1, a2, a3 = acc
                row = rbase + l
                return (a0 + rows_v[b][row, pl.ds(0, LANES)],
                        a1 + rows_v[b][row, pl.ds(LANES, LANES)],
                        a2 + rows_v[b][row, pl.ds(2 * LANES, LANES)],
                        a3 + rows_v[b][row, pl.ds(3 * LANES, LANES)])

            z = jnp.zeros((LANES,), jnp.float32)
            a0, a1, a2, a3 = lax.fori_loop(0, L, lbody, (z, z, z, z),
                                           unroll=5)
            pooled_v[b][r, pl.ds(0, LANES)] = a0 * scale
            pooled_v[b][r, pl.ds(LANES, LANES)] = a1 * scale
            pooled_v[b][r, pl.ds(2 * LANES, LANES)] = a2 * scale
            pooled_v[b][r, pl.ds(3 * LANES, LANES)] = a3 * scale
        pltpu.async_copy(pooled_v[b],
                         out_hbm.at[pl.ds(brow + c * CB, CB), pl.ds(col, EMB)],
                         sem_s[b])

    fire(0, 0)

    def body(c):
        fire(c + 1, 1)
        drain(c, 0)

        @pl.when(c + 2 < NCH)
        def _():
            fire(c + 2, 0)

        drain(c + 1, 1)

    pl.loop(0, NCH, step=2)(body)

    for b in range(2):
        pltpu.make_async_copy(pooled_v[b],
                              out_hbm.at[pl.ds(brow, CB), pl.ds(col, EMB)],
                              sem_s[b]).wait()


def _sc_pool(ids, table):
    mesh = plsc.VectorSubcoreMesh(core_axis_name="c", subcore_axis_name="s",
                                  num_cores=NCORES, num_subcores=NSUB)
    return pl.kernel(
        _sc_pool_body,
        out_type=jax.ShapeDtypeStruct((B, 2 * EMB), jnp.float32),
        mesh=mesh,
        compiler_params=pltpu.CompilerParams(use_tc_tiling_on_sc=False),
        scratch_types=[
            pltpu.VMEM((ROWS_PER_W * LP,), jnp.int32),
            pltpu.VMEM((CB * L, EMB), jnp.float32),
            pltpu.VMEM((CB * L, EMB), jnp.float32),
            pltpu.VMEM((CB, EMB), jnp.float32),
            pltpu.VMEM((CB, EMB), jnp.float32),
            pltpu.SemaphoreType.DMA,
            pltpu.SemaphoreType.DMA,
            pltpu.SemaphoreType.DMA,
            pltpu.SemaphoreType.DMA,
        ],
    )(ids, table)


VOCAB = 1000000
SPLIT = 512000
DEPAD_BN = 20480
DEPAD_G = SPLIT // DEPAD_BN
DEPAD_LAST = (VOCAB - 1) // DEPAD_BN


def _depad_body(ta_ref, tb_ref, out_ref):
    stacked = jnp.concatenate([ta_ref[...], tb_ref[...]], axis=0)
    out_ref[...] = stacked.T


def _depad(tableT):
    return pl.pallas_call(
        _depad_body,
        grid=(DEPAD_G,),
        in_specs=[
            pl.BlockSpec((EMB, DEPAD_BN), lambda i: (0, i)),
            pl.BlockSpec((EMB, DEPAD_BN),
                         lambda i: (0, jnp.minimum(i + DEPAD_G, DEPAD_LAST))),
        ],
        out_specs=pl.BlockSpec((DEPAD_BN, 2 * EMB), lambda i: (i, 0)),
        out_shape=jax.ShapeDtypeStruct((SPLIT, 2 * EMB), jnp.float32),
    )(tableT, tableT)


BM = 4096


def _mlp_body(ph_ref, w1t_ref, b1_ref, w2t_ref, b2_ref, out_ref):
    ph = ph_ref[...]
    p = ph[:, :EMB]
    h = ph[:, EMB:]
    f = jnp.concatenate([p, h, jnp.abs(p - h)], axis=1)
    x = jnp.dot(f, w1t_ref[...], preferred_element_type=jnp.float32)
    x = jnp.maximum(x + b1_ref[...], 0.0)
    logits = jnp.dot(x, w2t_ref[...], preferred_element_type=jnp.float32)
    logits = logits + b2_ref[...]
    m = jnp.max(logits, axis=1, keepdims=True)
    lse = m + jnp.log(jnp.sum(jnp.exp(logits - m), axis=1, keepdims=True))
    out_ref[...] = (logits - lse).T


def _mlp(pooled, w1t, b1, w2t, b2):
    grid = (B // BM,)
    return pl.pallas_call(
        _mlp_body,
        grid=grid,
        in_specs=[
            pl.BlockSpec((BM, 2 * EMB), lambda i: (i, 0)),
            pl.BlockSpec((3 * EMB, HID), lambda i: (0, 0)),
            pl.BlockSpec((1, HID), lambda i: (0, 0)),
            pl.BlockSpec((HID, 2), lambda i: (0, 0)),
            pl.BlockSpec((1, 2), lambda i: (0, 0)),
        ],
        out_specs=pl.BlockSpec((2, BM), lambda i: (0, i)),
        out_shape=jax.ShapeDtypeStruct((2, B), jnp.float32),
    )(pooled, w1t, b1, w2t, b2)


def kernel(premise, hypothesis, table, W1, b1, W2, b2):
    ids = jnp.concatenate([premise, hypothesis], axis=0).astype(jnp.int32)
    ids = jnp.where(ids < SPLIT, 2 * ids, 2 * (ids - SPLIT) + 1)
    ids = jnp.pad(ids, ((0, 0), (0, LP - L))).reshape(-1)
    table_lin = _depad(table.T).reshape(2 * SPLIT, EMB)
    pooled = _sc_pool(ids, table_lin)
    w1t = W1.T
    w2t = W2.T
    b1r = b1.reshape(1, HID)
    b2r = b2.reshape(1, 2)
    return _mlp(pooled, w1t, b1r, w2t, b2r).T
